# Initial kernel scaffold; baseline (speedup 1.0000x reference)
import jax
import jax.numpy as jnp
from jax.experimental import pallas as pl


def kernel(x, edge_index_1, edge_index_2, index_1, index_2, W11, b11, W12, b12, W21, b21, W22, b22, m1W1, m1b1, m1W2, m1b2, m2W1, m2b1, m2W2, m2b2, mW1, mb1, mW2, mb2):
    raise NotImplementedError("write your pallas kernel here")


# speedup vs baseline: 10.1529x; 10.1529x over previous
import functools

import jax
import jax.numpy as jnp
from jax import lax
from jax.experimental import pallas as pl
from jax.experimental.pallas import tpu as pltpu
from jax.experimental.pallas import tpu_sc as plsc

N = 10000
E = 320000
D_IN = 128
DIM = 256
HALF = 128
NSEG = 1000
NCLS = 7

NC, NS = 2, 16
CH = 128
NCHUNK = E // CH
NPAD = 10240

R = 1000
G = N // R

@functools.cache
def _sc_mesh():
  return plsc.VectorSubcoreMesh(
      core_axis_name="c", subcore_axis_name="s", num_cores=NC, num_subcores=NS)


def _fill(ref, n, value):
  for j in range(n // 16):
    ref[pl.ds(j * 16, 16)] = jnp.full((16,), value, ref.dtype)


def _deg_body(ei1, ei2, zn, degs, acc, dstv, onesv):
  c = lax.axis_index("c")
  s = lax.axis_index("s")
  _fill(onesv, CH, 1.0)

  @pl.when(s < 10)
  def _zero():
    pltpu.sync_copy(zn.at[pl.ds(s * 1024, 1024)], acc.at[pl.ds(s * 1024, 1024)])

  plsc.subcore_barrier()

  nmine = jnp.where(s < NCHUNK - 156 * NS, 157, 156)

  def run(ei):
    def chunk(i, _):
      base = pl.multiple_of((i * NS + s) * CH, CH)
      pltpu.sync_copy(ei.at[1].at[pl.ds(base, CH)], dstv)
      pltpu.sync_copy(onesv, acc.at[dstv], add=True)
      return ()
    lax.fori_loop(0, nmine, chunk, ())

  @pl.when(c == 0)
  def _run1():
    run(ei1)

  @pl.when(c == 1)
  def _run2():
    run(ei2)

  plsc.subcore_barrier()

  @pl.when(s < 10)
  def _flush():
    pltpu.sync_copy(acc.at[pl.ds(s * 1024, 1024)],
                    degs.at[pl.ds(c * NPAD + s * 1024, 1024)])


@functools.cache
def _deg_kernel():
  return pl.kernel(
    _deg_body,
    out_type=jax.ShapeDtypeStruct((2 * NPAD,), jnp.float32),
    mesh=_sc_mesh(),
    scratch_types=[
        pltpu.VMEM_SHARED((NPAD,), jnp.float32),
        pltpu.VMEM((CH,), jnp.int32),
        pltpu.VMEM((CH,), jnp.float32),
    ],
)


def _agg_body(p1, p2, ei1, ei2, znh, s1, s2, acc, srcv, dstv, rows, sem):
  c = lax.axis_index("c")
  s = lax.axis_index("s")
  off = c * N
  nmine = jnp.where(s < NCHUNK - 156 * NS, 157, 156)

  def zero():
    @pl.when(s < 10)
    def _():
      pltpu.sync_copy(znh.at[pl.ds(s * 1000, 1000)],
                      acc.at[pl.ds(s * 1000, 1000)])

  def accumulate(p, ei):
    def chunk(i, _):
      base = pl.multiple_of((i * NS + s) * CH, CH)
      pltpu.sync_copy(ei.at[0].at[pl.ds(base, CH)], srcv)
      for j in range(CH // 16):
        srcv[pl.ds(j * 16, 16)] = srcv[pl.ds(j * 16, 16)] + off
      pltpu.async_copy(p.at[srcv], rows, sem).wait()
      pltpu.sync_copy(ei.at[1].at[pl.ds(base, CH)], dstv)
      pltpu.sync_copy(rows, acc.at[dstv], add=True)
      return ()
    lax.fori_loop(0, nmine, chunk, ())

  def flush(dst):
    @pl.when(s < 10)
    def _():
      pltpu.sync_copy(acc.at[pl.ds(s * 1000, 1000)],
                      dst.at[pl.ds(off + s * 1000, 1000)])

  zero()
  plsc.subcore_barrier()
  accumulate(p1, ei1)
  plsc.subcore_barrier()
  flush(s1)
  plsc.subcore_barrier()
  zero()
  plsc.subcore_barrier()
  accumulate(p2, ei2)
  plsc.subcore_barrier()
  flush(s2)


@functools.cache
def _agg_kernel():
  return pl.kernel(
    _agg_body,
    out_type=(jax.ShapeDtypeStruct((2 * N, HALF), jnp.float32),
              jax.ShapeDtypeStruct((2 * N, HALF), jnp.float32)),
    mesh=_sc_mesh(),
    scratch_types=[
        pltpu.VMEM_SHARED((N, HALF), jnp.float32),
        pltpu.VMEM((CH,), jnp.int32),
        pltpu.VMEM((CH,), jnp.int32),
        pltpu.VMEM((CH, HALF), jnp.float32),
        pltpu.SemaphoreType.DMA,
    ],
)


_PCH = 128
_PNF = N // _PCH
_PTAIL = N - _PNF * _PCH

def _pool_body(h, idx1, idx2, znh, zn, s1, s2, cnt, acc1, acc2, cacc,
               rows, rows_t, iv1, iv2, iv1_t, iv2_t, onesv, onesv_t):
  c = lax.axis_index("c")
  s = lax.axis_index("s")
  _fill(onesv, _PCH, 1.0)
  _fill(onesv_t, _PTAIL, 1.0)

  @pl.when(s < 5)
  def _zero1():
    pltpu.sync_copy(znh.at[pl.ds(s * 200, 200)], acc1.at[pl.ds(s * 200, 200)])

  @pl.when((s >= 5) & (s < 10))
  def _zero2():
    pltpu.sync_copy(znh.at[pl.ds((s - 5) * 200, 200)],
                    acc2.at[pl.ds((s - 5) * 200, 200)])

  @pl.when(s == 10)
  def _zero_cnt():
    pltpu.sync_copy(zn.at[pl.ds(0, 1024)], cacc)

  plsc.subcore_barrier()

  count = jnp.where(s < _PNF - 4 * NS, 5, 4)

  def chunk(j, _):
    base = pl.multiple_of((j * NS + s) * _PCH, _PCH)
    pltpu.sync_copy(h.at[pl.ds(c * N + base, _PCH)], rows)
    pltpu.sync_copy(idx1.at[pl.ds(base, _PCH)], iv1)
    pltpu.sync_copy(idx2.at[pl.ds(base, _PCH)], iv2)
    pltpu.sync_copy(rows, acc1.at[iv1], add=True)
    pltpu.sync_copy(rows, acc2.at[iv2], add=True)

    @pl.when(c == 0)
    def _():
      pltpu.sync_copy(onesv, cacc.at[iv1], add=True)

    @pl.when(c == 1)
    def _():
      pltpu.sync_copy(onesv, cacc.at[iv2], add=True)
    return ()

  lax.fori_loop(0, count, chunk, ())

  @pl.when(s == 15)
  def _tail():
    base = _PNF * _PCH
    pltpu.sync_copy(h.at[pl.ds(c * N + base, _PTAIL)], rows_t)
    pltpu.sync_copy(idx1.at[pl.ds(base, _PTAIL)], iv1_t)
    pltpu.sync_copy(idx2.at[pl.ds(base, _PTAIL)], iv2_t)
    pltpu.sync_copy(rows_t, acc1.at[iv1_t], add=True)
    pltpu.sync_copy(rows_t, acc2.at[iv2_t], add=True)

    @pl.when(c == 0)
    def _():
      pltpu.sync_copy(onesv_t, cacc.at[iv1_t], add=True)

    @pl.when(c == 1)
    def _():
      pltpu.sync_copy(onesv_t, cacc.at[iv2_t], add=True)

  plsc.subcore_barrier()

  @pl.when(s < 5)
  def _flush1():
    pltpu.sync_copy(acc1.at[pl.ds(s * 200, 200)],
                    s1.at[pl.ds(c * NSEG + s * 200, 200)])

  @pl.when((s >= 5) & (s < 10))
  def _flush2():
    pltpu.sync_copy(acc2.at[pl.ds((s - 5) * 200, 200)],
                    s2.at[pl.ds(c * NSEG + (s - 5) * 200, 200)])

  @pl.when(s == 10)
  def _flushc():
    pltpu.sync_copy(cacc, cnt.at[pl.ds(c * 1024, 1024)])


@functools.cache
def _pool_kernel():
  return pl.kernel(
    _pool_body,
    out_type=(jax.ShapeDtypeStruct((2 * NSEG, HALF), jnp.float32),
              jax.ShapeDtypeStruct((2 * NSEG, HALF), jnp.float32),
              jax.ShapeDtypeStruct((2048,), jnp.float32)),
    mesh=_sc_mesh(),
    scratch_types=[
        pltpu.VMEM_SHARED((NSEG, HALF), jnp.float32),
        pltpu.VMEM_SHARED((NSEG, HALF), jnp.float32),
        pltpu.VMEM_SHARED((1024,), jnp.float32),
        pltpu.VMEM((_PCH, HALF), jnp.float32),
        pltpu.VMEM((_PTAIL, HALF), jnp.float32),
        pltpu.VMEM((_PCH,), jnp.int32),
        pltpu.VMEM((_PCH,), jnp.int32),
        pltpu.VMEM((_PTAIL,), jnp.int32),
        pltpu.VMEM((_PTAIL,), jnp.int32),
        pltpu.VMEM((_PCH,), jnp.float32),
        pltpu.VMEM((_PTAIL,), jnp.float32),
    ],
)


def _phase1_body(x, deg1, deg2, w11, w12, p11, p12, dinv1, dinv2):
  d1 = lax.rsqrt(deg1[...] + 1.0)
  d2 = lax.rsqrt(deg2[...] + 1.0)
  u11 = jnp.dot(x[...], w11[...], preferred_element_type=jnp.float32) * d1
  u12 = jnp.dot(x[...], w12[...], preferred_element_type=jnp.float32) * d2
  p11[0] = u11[:, :HALF]
  p11[1] = u11[:, HALF:]
  p12[0] = u12[:, :HALF]
  p12[1] = u12[:, HALF:]
  dinv1[...] = d1
  dinv2[...] = d2


def _tc_phase1(x, deg1, deg2, w11, w12):
  bw = lambda shape: pl.BlockSpec(shape, lambda i: (0,) * len(shape))
  return pl.pallas_call(
      _phase1_body,
      grid=(G,),
      in_specs=[
          pl.BlockSpec((R, D_IN), lambda i: (i, 0)),
          pl.BlockSpec((R, 1), lambda i: (i, 0)),
          pl.BlockSpec((R, 1), lambda i: (i, 0)),
          bw((D_IN, DIM)),
          bw((D_IN, DIM)),
      ],
      out_specs=[
          pl.BlockSpec((2, R, HALF), lambda i: (0, i, 0)),
          pl.BlockSpec((2, R, HALF), lambda i: (0, i, 0)),
          pl.BlockSpec((R, 1), lambda i: (i, 0)),
          pl.BlockSpec((R, 1), lambda i: (i, 0)),
      ],
      out_shape=[
          jax.ShapeDtypeStruct((2, N, HALF), jnp.float32),
          jax.ShapeDtypeStruct((2, N, HALF), jnp.float32),
          jax.ShapeDtypeStruct((N, 1), jnp.float32),
          jax.ShapeDtypeStruct((N, 1), jnp.float32),
      ],
  )(x, deg1, deg2, w11, w12)


def _mlp_front(s1, s2, p1, p2, d1, d2, b1, b2, mw1, mb1, mw2, mb2):
  x1 = jnp.concatenate([s1[0] + p1[0], s1[1] + p1[1]], axis=1)
  x1 = jnp.maximum(x1 * d1[...] + b1[...], 0.0)
  x2 = jnp.concatenate([s2[0] + p2[0], s2[1] + p2[1]], axis=1)
  x2 = jnp.maximum(x2 * d2[...] + b2[...], 0.0)
  h = jnp.concatenate([x1, x2], axis=1)
  t = jnp.maximum(
      jnp.dot(h, mw1[...], preferred_element_type=jnp.float32) + mb1[...], 0.0)
  return jnp.dot(t, mw2[...], preferred_element_type=jnp.float32) + mb2[...]


def _phase3_body(s11, s12, p11, p12, d1, d2, b11, b12, m1w1, m1b1, m1w2, m1b2,
                 w21, w22, p21, p22):
  h2 = _mlp_front(s11, s12, p11, p12, d1, d2, b11, b12, m1w1, m1b1, m1w2, m1b2)
  u21 = jnp.dot(h2, w21[...], preferred_element_type=jnp.float32) * d1[...]
  u22 = jnp.dot(h2, w22[...], preferred_element_type=jnp.float32) * d2[...]
  p21[0] = u21[:, :HALF]
  p21[1] = u21[:, HALF:]
  p22[0] = u22[:, :HALF]
  p22[1] = u22[:, HALF:]


def _phase5_body(s21, s22, p21, p22, d1, d2, b21, b22, m2w1, m2b1, m2w2, m2b2,
                 h3):
  out = _mlp_front(s21, s22, p21, p22, d1, d2, b21, b22, m2w1, m2b1, m2w2,
                   m2b2)
  h3[0] = out[:, :HALF]
  h3[1] = out[:, HALF:]


def _tc_phase35(body, n_out, s1, s2, p1, p2, d1, d2, b1, b2, mw1, mb1, mw2,
                mb2, w21=None, w22=None):
  bw = lambda shape: pl.BlockSpec(shape, lambda i: (0,) * len(shape))
  half = lambda: pl.BlockSpec((2, R, HALF), lambda i: (0, i, 0))
  col = lambda: pl.BlockSpec((R, 1), lambda i: (i, 0))
  in_specs = [half(), half(), half(), half(), col(), col(),
              bw((1, DIM)), bw((1, DIM)), bw((2 * DIM, DIM)), bw((1, DIM)),
              bw((DIM, DIM)), bw((1, DIM))]
  args = [s1, s2, p1, p2, d1, d2, b1, b2, mw1, mb1, mw2, mb2]
  if w21 is not None:
    in_specs += [bw((DIM, DIM)), bw((DIM, DIM))]
    args += [w21, w22]
  return pl.pallas_call(
      body,
      grid=(G,),
      in_specs=in_specs,
      out_specs=[half() for _ in range(n_out)],
      out_shape=[jax.ShapeDtypeStruct((2, N, HALF), jnp.float32)
                 for _ in range(n_out)],
  )(*args)


def _phase7_body(s1, s2, c1, c2, mw1, mb1, mw2p, mb2p, out):
  m1 = jnp.concatenate([s1[0], s1[1]], axis=1) / c1[...]
  m2 = jnp.concatenate([s2[0], s2[1]], axis=1) / c2[...]
  pooled = jnp.concatenate([m1, m2], axis=1)
  z = jnp.maximum(
      jnp.dot(pooled, mw1[...], preferred_element_type=jnp.float32) + mb1[...],
      0.0)
  o = jnp.dot(z, mw2p[...], preferred_element_type=jnp.float32) + mb2p[...]
  mx = jnp.max(o, axis=1, keepdims=True)
  lse = jnp.log(jnp.sum(jnp.exp(o - mx), axis=1, keepdims=True))
  out[...] = o - mx - lse


def _tc_phase7(s1, s2, c1, c2, mw1, mb1, mw2p, mb2p):
  bs = lambda shape: pl.BlockSpec(shape, lambda: (0,) * len(shape))
  return pl.pallas_call(
      _phase7_body,
      in_specs=[bs((2, NSEG, HALF)), bs((2, NSEG, HALF)), bs((NSEG, 1)),
                bs((NSEG, 1)), bs((2 * DIM, DIM)), bs((1, DIM)),
                bs((DIM, 128)), bs((1, 128))],
      out_specs=bs((NSEG, 128)),
      out_shape=jax.ShapeDtypeStruct((NSEG, 128), jnp.float32),
  )(s1, s2, c1, c2, mw1, mb1, mw2p, mb2p)


def kernel(x, edge_index_1, edge_index_2, index_1, index_2,
           W11, b11, W12, b12, W21, b21, W22, b22,
           m1W1, m1b1, m1W2, m1b2, m2W1, m2b1, m2W2, m2b2,
           mW1, mb1, mW2, mb2):
  zn = jnp.zeros((NPAD,), jnp.float32)
  znh = jnp.zeros((N, HALF), jnp.float32)

  degs = _deg_kernel()(edge_index_1, edge_index_2, zn)
  deg1 = degs[:N, None]
  deg2 = degs[NPAD:NPAD + N, None]

  p11, p12, dinv1, dinv2 = _tc_phase1(x, deg1, deg2, W11, W12)
  s11, s12 = _agg_kernel()(p11.reshape(2 * N, HALF), p12.reshape(2 * N, HALF),
                         edge_index_1, edge_index_2, znh)

  p21, p22 = _tc_phase35(
      _phase3_body, 2, s11.reshape(2, N, HALF), s12.reshape(2, N, HALF),
      p11, p12, dinv1, dinv2, b11[None, :], b12[None, :], m1W1,
      m1b1[None, :], m1W2, m1b2[None, :], W21, W22)

  s21, s22 = _agg_kernel()(p21.reshape(2 * N, HALF), p22.reshape(2 * N, HALF),
                         edge_index_1, edge_index_2, znh)

  h3 = _tc_phase35(
      _phase5_body, 1, s21.reshape(2, N, HALF), s22.reshape(2, N, HALF),
      p21, p22, dinv1, dinv2, b21[None, :], b22[None, :], m2W1,
      m2b1[None, :], m2W2, m2b2[None, :])[0]

  ps1, ps2, cnt = _pool_kernel()(h3.reshape(2 * N, HALF), index_1, index_2,
                               znh[:2 * NSEG], zn)
  c1 = jnp.clip(cnt[:NSEG], 1.0)[:, None]
  c2 = jnp.clip(cnt[1024:1024 + NSEG], 1.0)[:, None]

  mw2p = jnp.pad(mW2, ((0, 0), (0, 128 - NCLS)))
  mb2p = jnp.pad(mb2, (0, 128 - NCLS), constant_values=-1e9)[None, :]
  out = _tc_phase7(ps1.reshape(2, NSEG, HALF), ps2.reshape(2, NSEG, HALF),
                   c1, c2, mW1, mb1[None, :], mw2p, mb2p)
  return out[:, :NCLS]
